# baseline (speedup 1.0000x reference)
import functools

import jax
import jax.numpy as jnp
from jax import lax
from jax.experimental import pallas as pl
from jax.experimental.pallas import tpu as pltpu
from jax.experimental.pallas import tpu_sc as plsc

_info = plsc.get_sparse_core_info()
_NC, _NS, _NL = _info.num_cores, _info.num_subcores, _info.num_lanes
_NW = _NC * _NS

_TAIL = 16


def _make_gather(n_batch: int, n_seq: int, n_tab: int, d: int):
    n_rows = n_batch * n_seq
    rows_per_w = n_rows // _NW
    w_per_batch = _NW // n_batch
    chunk = _NL
    n_chunks = rows_per_w // chunk
    tail0 = n_tab - _TAIL
    mesh = plsc.VectorSubcoreMesh(core_axis_name="c", subcore_axis_name="s")

    @functools.partial(
        pl.kernel,
        mesh=mesh,
        out_type=jax.ShapeDtypeStruct((n_batch, n_seq, d), jnp.float32),
        scratch_types=[
            pltpu.VMEM((n_chunks, chunk), jnp.int32),
            pltpu.VMEM((1, chunk), jnp.int32),
            pltpu.VMEM((1, chunk), jnp.int32),
            pltpu.VMEM((_TAIL, d), jnp.float32),
        ] + [pltpu.VMEM((chunk, d), jnp.float32)] * 6
          + [pltpu.SemaphoreType.DMA] * 12,
    )
    def gather_kernel(table_hbm, tail_hbm, idxc_hbm, idxr_hbm, gates_hbm,
                      out_hbm, idx_clp, idxr_v, g_v, tl, *bufs):
        rows = bufs[:6]
        gsem = bufs[6:12]
        ssem = bufs[12:18]
        wid = lax.axis_index("s") * _NC + lax.axis_index("c")
        batch = wid // w_per_batch
        base = (wid % w_per_batch) * rows_per_w

        pltpu.sync_copy(idxc_hbm.at[wid], idx_clp)
        pltpu.sync_copy(gates_hbm.at[pl.ds(wid, 1)], g_v)
        pltpu.sync_copy(tail_hbm, tl)

        NB = 6

        def gather(c):
            return pltpu.async_copy(
                table_hbm.at[0].at[idx_clp.at[c]], rows[c % NB],
                gsem[c % NB])

        def store(c):
            return pltpu.async_copy(
                rows[c % NB],
                out_hbm.at[batch, pl.ds(base + c * chunk, chunk)],
                ssem[c % NB])

        LOOKAHEAD = 3
        pend_g = {c: gather(c) for c in range(LOOKAHEAD)}
        pend_s = {}
        for c in range(n_chunks):
            nxt = c + LOOKAHEAD
            if nxt < n_chunks:
                prev = nxt - NB
                if prev in pend_s:
                    pend_s.pop(prev).wait()
                pend_g[nxt] = gather(nxt)
            pend_g.pop(c).wait()
            pend_s[c] = store(c)
        for c in sorted(pend_s):
            pend_s[c].wait()

        mask = g_v[0, :][0]

        @pl.when(mask != 0)
        def _():
            @pl.loop(0, n_chunks)
            def _(c):
                @pl.when(((mask >> c) & 1) != 0)
                def _():
                    pltpu.sync_copy(idxr_hbm.at[wid].at[pl.ds(c, 1)], idxr_v)
                    idxv = idxr_v[0, :]
                    for r in range(chunk):
                        iv = idxv[r]

                        @pl.when(iv >= tail0)
                        def _():
                            pltpu.sync_copy(
                                tl.at[pl.ds(iv - tail0, 1)],
                                out_hbm.at[batch,
                                           pl.ds(base + c * chunk + r, 1)])

    return gather_kernel


def kernel(x_len, index, pe):
    if index is None:
        return pe[:, :x_len, :]
    b, s, _ = index.shape
    n = b * s
    n_tab = pe.shape[1]
    tail0 = n_tab - _TAIL
    tail = pe[0, tail0:, :]
    idx_3d = index.reshape(_NW, n // (_NW * _NL), _NL).astype(jnp.int32)
    idx_clp = jnp.minimum(idx_3d, tail0 - 1)
    is_tail = jnp.any(idx_3d >= tail0, axis=2)
    bits = jnp.sum(
        jnp.where(is_tail, jnp.int32(1) << jnp.arange(
            idx_3d.shape[1], dtype=jnp.int32)[None, :], 0),
        axis=1, dtype=jnp.int32)
    gates = jnp.concatenate(
        [bits[:, None], jnp.zeros((_NW, _NL - 1), jnp.int32)], axis=1)
    return _make_gather(b, s, n_tab, pe.shape[2])(
        pe, tail, idx_clp, idx_3d, gates)


# speedup vs baseline: 1.0354x; 1.0354x over previous
import functools

import jax
import jax.numpy as jnp
from jax import lax
from jax.experimental import pallas as pl
from jax.experimental.pallas import tpu as pltpu
from jax.experimental.pallas import tpu_sc as plsc

_info = plsc.get_sparse_core_info()
_NC, _NS, _NL = _info.num_cores, _info.num_subcores, _info.num_lanes
_NW = _NC * _NS

_TAIL = 16


def _make_gather(n_batch: int, n_seq: int, n_tab: int, d: int):
    n_rows = n_batch * n_seq
    rows_per_w = n_rows // _NW
    w_per_batch = _NW // n_batch
    chunk = _NL
    n_chunks = rows_per_w // chunk
    tail0 = n_tab - _TAIL
    mesh = plsc.VectorSubcoreMesh(core_axis_name="c", subcore_axis_name="s")

    @functools.partial(
        pl.kernel,
        mesh=mesh,
        compiler_params=pltpu.CompilerParams(needs_layout_passes=False),
        out_type=jax.ShapeDtypeStruct((n_batch, n_seq, d), jnp.float32),
        scratch_types=[
            pltpu.VMEM((n_chunks, chunk), jnp.int32),
            pltpu.VMEM((n_chunks, chunk), jnp.int32),
            pltpu.VMEM((_TAIL, d), jnp.float32),
        ] + [pltpu.VMEM((chunk, d), jnp.float32)] * 6
          + [pltpu.SemaphoreType.DMA] * 12,
    )
    def gather_kernel(table_hbm, tail_hbm, idx_hbm, out_hbm, idx_v, idx_clp,
                      tl, *bufs):
        rows = bufs[:6]
        gsem = bufs[6:12]
        ssem = bufs[12:18]
        wid = lax.axis_index("s") * _NC + lax.axis_index("c")
        batch = wid // w_per_batch
        base = (wid % w_per_batch) * rows_per_w

        pltpu.sync_copy(idx_hbm.at[wid], idx_v)
        pltpu.sync_copy(tail_hbm, tl)

        @pl.loop(0, n_chunks)
        def _(c):
            idx_clp[c, :] = jnp.minimum(idx_v[c, :], tail0 - 1)

        NB = 6

        def gather(c):
            return pltpu.async_copy(
                table_hbm.at[0].at[idx_clp.at[c]], rows[c % NB],
                gsem[c % NB])

        def store(c):
            return pltpu.async_copy(
                rows[c % NB],
                out_hbm.at[batch, pl.ds(base + c * chunk, chunk)],
                ssem[c % NB])

        LOOKAHEAD = 3
        pend_g = {c: gather(c) for c in range(LOOKAHEAD)}
        pend_s = {}
        for c in range(n_chunks):
            nxt = c + LOOKAHEAD
            if nxt < n_chunks:
                prev = nxt - NB
                if prev in pend_s:
                    pend_s.pop(prev).wait()
                pend_g[nxt] = gather(nxt)
            pend_g.pop(c).wait()
            pend_s[c] = store(c)
        for c in sorted(pend_s):
            pend_s[c].wait()

        @pl.loop(0, n_chunks)
        def _(c):
            idxv = idx_v[c, :]
            cnt = plsc.all_reduce_population_count(idxv >= tail0)

            @pl.when(cnt[0] > 0)
            def _():
                for r in range(chunk):
                    iv = idxv[r]

                    @pl.when(iv >= tail0)
                    def _():
                        pltpu.sync_copy(
                            tl.at[pl.ds(iv - tail0, 1)],
                            out_hbm.at[batch,
                                       pl.ds(base + c * chunk + r, 1)])

    return gather_kernel


def kernel(x_len, index, pe):
    if index is None:
        return pe[:, :x_len, :]
    b, s, _ = index.shape
    n = b * s
    n_tab = pe.shape[1]
    tail = pe[0, n_tab - _TAIL:, :]
    idx_3d = index.reshape(_NW, n // (_NW * _NL), _NL).astype(jnp.int32)
    return _make_gather(b, s, n_tab, pe.shape[2])(pe, tail, idx_3d)
